# baseline (speedup 1.0000x reference)
import functools

import jax
import jax.numpy as jnp
from jax import lax
from jax.experimental import pallas as pl
from jax.experimental.pallas import tpu as pltpu
from jax.experimental.pallas import tpu_sc as plsc

_NUM_CORES = 2
_NUM_SUBCORES = 16
_NUM_WORKERS = _NUM_CORES * _NUM_SUBCORES
_IDX_CHUNK = 128


def _tc_flatten(theta_t, dim, copy_users, stride):
    def body(in_ref, out_ref, sem):
        copies = [
            pltpu.async_copy(
                in_ref.at[d, pl.ds(0, copy_users)],
                out_ref.at[pl.ds(d * stride, copy_users)],
                sem,
            )
            for d in range(dim)
        ]
        for c in copies:
            c.wait()

    return pl.pallas_call(
        body,
        in_specs=[pl.BlockSpec(memory_space=pl.ANY)],
        out_specs=pl.BlockSpec(memory_space=pl.ANY),
        out_shape=jax.ShapeDtypeStruct((dim * stride,), jnp.float32),
        scratch_shapes=[pltpu.SemaphoreType.DMA],
    )(theta_t)


def _sc_gather_elements(theta_flat, idx3, dim, batch):
    b_per_w = batch // _NUM_WORKERS
    per_d = b_per_w // _IDX_CHUNK
    k_streams = dim * per_d
    mesh = plsc.VectorSubcoreMesh(core_axis_name="c", subcore_axis_name="s")

    @functools.partial(
        pl.kernel,
        mesh=mesh,
        out_type=jax.ShapeDtypeStruct((dim, batch), jnp.float32),
        scratch_types=[
            pltpu.VMEM((k_streams, _IDX_CHUNK), jnp.int32),
            pltpu.VMEM((dim, b_per_w), jnp.float32),
            pltpu.SemaphoreType.DMA,
        ],
    )
    def gather_kernel(flat_hbm, idx_hbm, out_hbm, idx_v, gt_v, sem):
        wid = lax.axis_index("s") * _NUM_CORES + lax.axis_index("c")
        pltpu.sync_copy(idx_hbm.at[wid], idx_v)
        copies = []
        for k in range(k_streams):
            copies.append(
                pltpu.async_copy(
                    flat_hbm.at[idx_v.at[k]],
                    gt_v.at[k // per_d, pl.ds((k % per_d) * _IDX_CHUNK,
                                              _IDX_CHUNK)],
                    sem,
                )
            )
        for c in copies:
            c.wait()
        pltpu.sync_copy(gt_v, out_hbm.at[:, pl.ds(wid * b_per_w, b_per_w)])

    return gather_kernel(theta_flat, idx3)


def _tc_utility_logsoftmax(gt, sel, tail_t, alpha_item, batch, num_items,
                           dim, tail):
    blk = 1024

    def body(gt_ref, sel_ref, tail_ref, alpha_ref, out_ref):
        g = gt_ref[...]
        s_row = sel_ref[...]
        is_tail = s_row >= 0
        onehot = (
            lax.broadcasted_iota(jnp.int32, (tail, blk), 0) == s_row
        ).astype(jnp.float32)
        corr = lax.dot_general(
            tail_ref[...], onehot, (((1,), (0,)), ((), ())),
            preferred_element_type=jnp.float32,
        )
        g = jnp.where(is_tail, corr, g)
        u = lax.dot_general(
            g, alpha_ref[...], (((0,), (1,)), ((), ())),
            preferred_element_type=jnp.float32,
        )
        m = jnp.max(u, axis=-1, keepdims=True)
        e = jnp.exp(u - m)
        s = jnp.sum(e, axis=-1, keepdims=True)
        out_ref[...] = u - m - jnp.log(s)

    return pl.pallas_call(
        body,
        grid=(batch // blk,),
        in_specs=[
            pl.BlockSpec((dim, blk), lambda i: (0, i)),
            pl.BlockSpec((1, blk), lambda i: (0, i)),
            pl.BlockSpec((dim, tail), lambda i: (0, 0)),
            pl.BlockSpec((num_items, dim), lambda i: (0, 0)),
        ],
        out_specs=pl.BlockSpec((blk, num_items), lambda i: (i, 0)),
        out_shape=jax.ShapeDtypeStruct((batch, num_items), jnp.float32),
    )(gt, sel, tail_t, alpha_item)


def kernel(user_index, theta_user, alpha_item):
    batch = user_index.shape[0]
    num_users, dim = theta_user.shape
    num_items = alpha_item.shape[0]
    b_per_w = batch // _NUM_WORKERS

    copy_users = (num_users // 1024) * 1024
    stride = copy_users + 1024
    tail = num_users - copy_users

    idx = user_index.astype(jnp.int32)
    safe_u = jnp.where(idx < copy_users, idx, 0)
    u_r = safe_u.reshape(_NUM_WORKERS, 1, b_per_w)
    d_off = (jnp.arange(dim, dtype=jnp.int32) * stride).reshape(1, dim, 1)
    idx3 = (u_r + d_off).reshape(
        _NUM_WORKERS, dim * b_per_w // _IDX_CHUNK, _IDX_CHUNK)
    sel = jnp.where(idx < copy_users, -1, idx - copy_users).reshape(1, batch)
    tail_t = theta_user[copy_users:].T

    theta_t = theta_user.T
    theta_flat = _tc_flatten(theta_t, dim, copy_users, stride)
    gt = _sc_gather_elements(theta_flat, idx3, dim, batch)
    return _tc_utility_logsoftmax(gt, sel, tail_t, alpha_item, batch,
                                  num_items, dim, tail)


# speedup vs baseline: 6.6362x; 6.6362x over previous
import functools

import jax
import jax.numpy as jnp
from jax import lax
from jax.experimental import pallas as pl
from jax.experimental.pallas import tpu as pltpu
from jax.experimental.pallas import tpu_sc as plsc

_NUM_CORES = 2
_NUM_SUBCORES = 16
_NUM_WORKERS = _NUM_CORES * _NUM_SUBCORES
_IDX_CHUNK = 128


def _sc_gather(theta_user, idx2d, batch, dim):
    b_per_w = batch // _NUM_WORKERS
    chunks = b_per_w // _IDX_CHUNK
    mesh = plsc.VectorSubcoreMesh(core_axis_name="c", subcore_axis_name="s")

    @functools.partial(
        pl.kernel,
        mesh=mesh,
        compiler_params=pltpu.CompilerParams(use_tc_tiling_on_sc=False),
        out_type=jax.ShapeDtypeStruct((batch, dim), jnp.float32),
        scratch_types=[
            pltpu.VMEM((chunks, _IDX_CHUNK), jnp.int32),
            pltpu.VMEM((b_per_w, dim), jnp.float32),
            pltpu.SemaphoreType.DMA,
        ],
    )
    def gather_kernel(theta_hbm, idx_hbm, out_hbm, idx_v, rows_v, sem):
        wid = lax.axis_index("s") * _NUM_CORES + lax.axis_index("c")
        base = wid * b_per_w
        pltpu.sync_copy(idx_hbm.at[pl.ds(wid * chunks, chunks)], idx_v)
        copies = []
        for j in range(chunks):
            copies.append(
                pltpu.async_copy(
                    theta_hbm.at[idx_v.at[j]],
                    rows_v.at[pl.ds(j * _IDX_CHUNK, _IDX_CHUNK)],
                    sem,
                )
            )
        for c in copies:
            c.wait()
        pltpu.sync_copy(rows_v, out_hbm.at[pl.ds(base, b_per_w)])

    return gather_kernel(theta_user, idx2d)


def _tc_utility_logsoftmax(theta, alpha_item, batch, num_items, dim):
    blk = 2048

    def body(theta_ref, alpha_ref, out_ref):
        th = theta_ref[...]
        al = alpha_ref[...]
        u = lax.dot_general(
            th, al, (((1,), (1,)), ((), ())), preferred_element_type=jnp.float32
        )
        m = jnp.max(u, axis=-1, keepdims=True)
        e = jnp.exp(u - m)
        s = jnp.sum(e, axis=-1, keepdims=True)
        out_ref[...] = u - m - jnp.log(s)

    return pl.pallas_call(
        body,
        grid=(batch // blk,),
        in_specs=[
            pl.BlockSpec((blk, dim), lambda i: (i, 0)),
            pl.BlockSpec((num_items, dim), lambda i: (0, 0)),
        ],
        out_specs=pl.BlockSpec((blk, num_items), lambda i: (i, 0)),
        out_shape=jax.ShapeDtypeStruct((batch, num_items), jnp.float32),
    )(theta, alpha_item)


def kernel(user_index, theta_user, alpha_item):
    batch = user_index.shape[0]
    num_items, dim = alpha_item.shape
    idx2d = user_index.astype(jnp.int32).reshape(batch // _IDX_CHUNK, _IDX_CHUNK)
    theta = _sc_gather(theta_user, idx2d, batch, dim)
    return _tc_utility_logsoftmax(theta, alpha_item, batch, num_items, dim)


# speedup vs baseline: 6.8145x; 1.0269x over previous
import functools

import jax
import jax.numpy as jnp
from jax import lax
from jax.experimental import pallas as pl
from jax.experimental.pallas import tpu as pltpu
from jax.experimental.pallas import tpu_sc as plsc

_NUM_CORES = 2
_NUM_SUBCORES = 16
_NUM_WORKERS = _NUM_CORES * _NUM_SUBCORES
_IDX_CHUNK = 128


def _sc_gather_pad(theta_pad, idx2d, batch):
    dim = 128
    b_per_w = batch // _NUM_WORKERS
    chunks = b_per_w // _IDX_CHUNK
    mesh = plsc.VectorSubcoreMesh(core_axis_name="c", subcore_axis_name="s")

    @functools.partial(
        pl.kernel,
        mesh=mesh,
        out_type=jax.ShapeDtypeStruct((batch, dim), jnp.float32),
        scratch_types=[
            pltpu.VMEM((chunks, _IDX_CHUNK), jnp.int32),
            pltpu.VMEM((b_per_w, dim), jnp.float32),
            pltpu.SemaphoreType.DMA,
        ],
    )
    def gather_kernel(theta_hbm, idx_hbm, out_hbm, idx_v, rows_v, sem):
        wid = lax.axis_index("s") * _NUM_CORES + lax.axis_index("c")
        base = wid * b_per_w
        pltpu.sync_copy(idx_hbm.at[pl.ds(wid * chunks, chunks)], idx_v)
        copies = []
        for j in range(chunks):
            copies.append(
                pltpu.async_copy(
                    theta_hbm.at[idx_v.at[j]],
                    rows_v.at[pl.ds(j * _IDX_CHUNK, _IDX_CHUNK)],
                    sem,
                )
            )
        for c in copies:
            c.wait()
        pltpu.sync_copy(rows_v, out_hbm.at[pl.ds(base, b_per_w)])

    return gather_kernel(theta_pad, idx2d)


def _tc_utility_logsoftmax(theta, alpha_item, batch, num_items, dim):
    blk = 2048

    def body(theta_ref, alpha_ref, out_ref):
        th = theta_ref[...]
        al = alpha_ref[...]
        u = lax.dot_general(
            th, al, (((1,), (1,)), ((), ())), preferred_element_type=jnp.float32
        )
        m = jnp.max(u, axis=-1, keepdims=True)
        e = jnp.exp(u - m)
        s = jnp.sum(e, axis=-1, keepdims=True)
        out_ref[...] = u - m - jnp.log(s)

    return pl.pallas_call(
        body,
        grid=(batch // blk,),
        in_specs=[
            pl.BlockSpec((blk, dim), lambda i: (i, 0)),
            pl.BlockSpec((num_items, dim), lambda i: (0, 0)),
        ],
        out_specs=pl.BlockSpec((blk, num_items), lambda i: (i, 0)),
        out_shape=jax.ShapeDtypeStruct((batch, num_items), jnp.float32),
    )(theta, alpha_item)


def kernel(user_index, theta_user, alpha_item):
    batch = user_index.shape[0]
    num_items, dim = alpha_item.shape
    idx2d = user_index.astype(jnp.int32).reshape(batch // _IDX_CHUNK, _IDX_CHUNK)
    theta_pad = jnp.pad(theta_user, ((0, 0), (0, 128 - dim)))
    alpha_pad = jnp.pad(alpha_item, ((0, 0), (0, 128 - dim)))
    theta = _sc_gather_pad(theta_pad, idx2d, batch)
    return _tc_utility_logsoftmax(theta, alpha_pad, batch, num_items, 128)
